# baseline (device time: 64019 ns/iter reference)
import jax
import jax.numpy as jnp
from jax import lax
from jax.experimental import pallas as pl
from jax.experimental.pallas import tpu as pltpu

N_DEV = 4
B = 2
SQ = 512
SKV_SHARD = 512
HQ = 8
DH = 64
BH = B * HQ
C = 4
GRP = BH // C


def kernel(x, Wq, K_ext, V_ext, Wo):
    d_model = x.shape[-1]
    K_ext = K_ext.reshape(B, SKV_SHARD, HQ * DH)
    V_ext = V_ext.reshape(B, SKV_SHARD, HQ * DH)

    def body(
        x_ref, wq_ref, k_ref, v_ref, wo_ref, out_ref,
        ctx_comm, stats_comm, macc_ref, lacc_ref, ctx_acc,
        ctx_send_sems, ctx_recv_sems, st_send_sems, st_recv_sems,
    ):
        my_pos = lax.axis_index("i")
        left = (my_pos - 1) % N_DEV
        right = (my_pos + 1) % N_DEV

        barrier_sem = pltpu.get_barrier_semaphore()
        for nbr in [left, right]:
            pl.semaphore_signal(
                barrier_sem, inc=1,
                device_id=(nbr,), device_id_type=pl.DeviceIdType.MESH,
            )
        pl.semaphore_wait(barrier_sem, 2)

        started = []

        def make_chunk_rdmas(h, c):
            rc = pltpu.make_async_remote_copy(
                src_ref=ctx_comm.at[h, pl.ds(GRP * c, GRP)],
                dst_ref=ctx_comm.at[h + 1, pl.ds(GRP * c, GRP)],
                send_sem=ctx_send_sems.at[h, c],
                recv_sem=ctx_recv_sems.at[h, c],
                device_id=(right,),
                device_id_type=pl.DeviceIdType.MESH,
            )
            rs = pltpu.make_async_remote_copy(
                src_ref=stats_comm.at[h, c],
                dst_ref=stats_comm.at[h + 1, c],
                send_sem=st_send_sems.at[h, c],
                recv_sem=st_recv_sems.at[h, c],
                device_id=(right,),
                device_id_type=pl.DeviceIdType.MESH,
            )
            return rc, rs

        def start_hop(h, c):
            rc, rs = make_chunk_rdmas(h, c)
            rc.start()
            rs.start()
            started.extend((rc, rs))

        def wait_arrival(h, c):
            rc, rs = make_chunk_rdmas(h, c)
            rc.wait_recv()
            rs.wait_recv()

        def combine(slot, c):
            rows = pl.ds(GRP * c, GRP)
            m_in = stats_comm[slot, c, 0:GRP, :]
            l_in = stats_comm[slot, c, GRP:2 * GRP, :]
            m_old = macc_ref[rows, :]
            m_new = jnp.maximum(m_old, m_in)
            a = jnp.exp(m_old - m_new)
            g = jnp.exp(m_in - m_new)
            lacc_ref[rows, :] = a * lacc_ref[rows, :] + g * l_in
            ctx_acc[rows] = (
                a[:, None, :] * ctx_acc[rows]
                + g[:, None, :] * ctx_comm[slot, rows].astype(jnp.float32)
            )
            macc_ref[rows, :] = m_new

        kv_off = my_pos * SKV_SHARD
        qi = lax.broadcasted_iota(jnp.int32, (SQ, SKV_SHARD), 0)
        kj = lax.broadcasted_iota(jnp.int32, (SQ, SKV_SHARD), 1) + kv_off
        mask = (jnp.abs(qi - kj) <= 128) | (kj < 32) | (qi < 32)

        for bh in range(BH):
            b, h = divmod(bh, HQ)
            hs = slice(h * DH, (h + 1) * DH)
            if h == 0:
                q_all = jnp.dot(
                    x_ref[b].astype(jnp.bfloat16),
                    wq_ref[...].astype(jnp.bfloat16),
                    preferred_element_type=jnp.float32,
                ).astype(jnp.bfloat16)
                k_all = k_ref[b].astype(jnp.bfloat16)
                v_all = v_ref[b].astype(jnp.bfloat16)
            q = q_all[:, hs]
            k = k_all[:, hs]
            v = v_all[:, hs]
            s = lax.dot_general(
                q, k, (((1,), (1,)), ((), ())),
                preferred_element_type=jnp.float32,
            ) * 0.125
            s = jnp.where(mask, s, -1e9)
            m = jnp.max(s, axis=1)
            p = jnp.exp(s - m[:, None])
            l = jnp.sum(p, axis=1)
            ctx_t = lax.dot_general(
                v, p.astype(jnp.bfloat16), (((0,), (1,)), ((), ())),
                preferred_element_type=jnp.float32,
            )
            c, r = divmod(bh, GRP)
            ctx_comm[0, bh] = ctx_t.astype(jnp.bfloat16)
            stats_comm[0, c, r, :] = m
            stats_comm[0, c, GRP + r, :] = l
            ctx_acc[bh] = ctx_t
            macc_ref[bh, :] = m
            lacc_ref[bh, :] = l
            if r == GRP - 1:
                start_hop(0, c)
                if c > 0:
                    wait_arrival(0, c - 1)
                    start_hop(1, c - 1)
                    combine(1, c - 1)

        wait_arrival(0, C - 1)
        start_hop(1, C - 1)
        combine(1, C - 1)
        for c in range(C):
            wait_arrival(1, c)
            start_hop(2, c)
            combine(2, c)
        for c in range(C):
            wait_arrival(2, c)
            combine(3, c)

        wo_bf = wo_ref[...].astype(jnp.bfloat16)
        for b in range(B):
            rows = pl.ds(b * HQ, HQ)
            l_b = lacc_ref[rows, :]
            ctxn = ctx_acc[rows] / l_b[:, None, :]
            cc = ctxn.astype(jnp.bfloat16).reshape(HQ * DH, SQ)
            out_ref[b] = lax.dot_general(
                cc, wo_bf, (((0,), (0,)), ((), ())),
                preferred_element_type=jnp.float32,
            )

        for rdma in started:
            rdma.wait_send()

    return pl.pallas_call(
        body,
        out_shape=jax.ShapeDtypeStruct((B, SQ, d_model), jnp.float32),
        in_specs=[pl.BlockSpec(memory_space=pltpu.VMEM)] * 5,
        out_specs=pl.BlockSpec(memory_space=pltpu.VMEM),
        scratch_shapes=[
            pltpu.VMEM((N_DEV, BH, DH, SQ), jnp.bfloat16),
            pltpu.VMEM((N_DEV, C, 2 * GRP, SQ), jnp.float32),
            pltpu.VMEM((BH, SQ), jnp.float32),
            pltpu.VMEM((BH, SQ), jnp.float32),
            pltpu.VMEM((BH, DH, SQ), jnp.float32),
            pltpu.SemaphoreType.DMA((N_DEV - 1, C)),
            pltpu.SemaphoreType.DMA((N_DEV - 1, C)),
            pltpu.SemaphoreType.DMA((N_DEV - 1, C)),
            pltpu.SemaphoreType.DMA((N_DEV - 1, C)),
        ],
        compiler_params=pltpu.CompilerParams(collective_id=0),
    )(x, Wq, K_ext, V_ext, Wo)


# device time: 52885 ns/iter; 1.2105x vs baseline; 1.2105x over previous
import jax
import jax.numpy as jnp
from jax import lax
from jax.experimental import pallas as pl
from jax.experimental.pallas import tpu as pltpu

N_DEV = 4
B = 2
SQ = 512
SKV_SHARD = 512
HQ = 8
DH = 64
BH = B * HQ
C = 4
GRP = BH // C


def kernel(x, Wq, K_ext, V_ext, Wo):
    d_model = x.shape[-1]
    K_ext = K_ext.reshape(B, SKV_SHARD, HQ * DH)
    V_ext = V_ext.reshape(B, SKV_SHARD, HQ * DH)

    def body(
        x_ref, wq_ref, k_ref, v_ref, wo_ref, out_ref,
        ctx_comm, stats_comm, lacc_ref, ctx_acc,
        ctx_send_sems, ctx_recv_sems, st_send_sems, st_recv_sems,
    ):
        my_pos = lax.axis_index("i")
        left = (my_pos - 1) % N_DEV
        right = (my_pos + 1) % N_DEV

        barrier_sem = pltpu.get_barrier_semaphore()
        for nbr in [left, right]:
            pl.semaphore_signal(
                barrier_sem, inc=1,
                device_id=(nbr,), device_id_type=pl.DeviceIdType.MESH,
            )
        pl.semaphore_wait(barrier_sem, 2)

        started = []

        def make_chunk_rdmas(h, c):
            rc = pltpu.make_async_remote_copy(
                src_ref=ctx_comm.at[h, pl.ds(GRP * c, GRP)],
                dst_ref=ctx_comm.at[h + 1, pl.ds(GRP * c, GRP)],
                send_sem=ctx_send_sems.at[h, c],
                recv_sem=ctx_recv_sems.at[h, c],
                device_id=(right,),
                device_id_type=pl.DeviceIdType.MESH,
            )
            rs = pltpu.make_async_remote_copy(
                src_ref=stats_comm.at[h, c],
                dst_ref=stats_comm.at[h + 1, c],
                send_sem=st_send_sems.at[h, c],
                recv_sem=st_recv_sems.at[h, c],
                device_id=(right,),
                device_id_type=pl.DeviceIdType.MESH,
            )
            return rc, rs

        def start_hop(h, c):
            rc, rs = make_chunk_rdmas(h, c)
            rc.start()
            rs.start()
            started.extend((rc, rs))

        def wait_arrival(h, c):
            rc, rs = make_chunk_rdmas(h, c)
            rc.wait_recv()
            rs.wait_recv()

        def combine(slot, c):
            rows = pl.ds(GRP * c, GRP)
            lacc_ref[rows, :] = lacc_ref[rows, :] + stats_comm[slot, c]
            ctx_acc[rows] = (
                ctx_acc[rows] + ctx_comm[slot, rows].astype(jnp.float32)
            )

        kv_off = my_pos * SKV_SHARD
        qi = lax.broadcasted_iota(jnp.int32, (SQ, SKV_SHARD), 0)
        kj = lax.broadcasted_iota(jnp.int32, (SQ, SKV_SHARD), 1) + kv_off
        mask = (jnp.abs(qi - kj) <= 128) | (kj < 32) | (qi < 32)
        bias = jnp.where(mask, 0.0, -1e9).astype(jnp.float32)

        for bh in range(BH):
            b, h = divmod(bh, HQ)
            hs = slice(h * DH, (h + 1) * DH)
            if h == 0:
                q_all = (jnp.dot(
                    x_ref[b].astype(jnp.bfloat16),
                    wq_ref[...].astype(jnp.bfloat16),
                    preferred_element_type=jnp.float32,
                ) * 0.125).astype(jnp.bfloat16)
                k_all = k_ref[b].astype(jnp.bfloat16)
                v_all = v_ref[b].astype(jnp.bfloat16)
            q = q_all[:, hs]
            k = k_all[:, hs]
            v = v_all[:, hs]
            s = lax.dot_general(
                q, k, (((1,), (1,)), ((), ())),
                preferred_element_type=jnp.float32,
            )
            p = jnp.exp(s + bias)
            l = jnp.sum(p, axis=1)
            ctx_t = lax.dot_general(
                v, p.astype(jnp.bfloat16), (((0,), (1,)), ((), ())),
                preferred_element_type=jnp.float32,
            )
            c, r = divmod(bh, GRP)
            ctx_comm[0, bh] = ctx_t.astype(jnp.bfloat16)
            stats_comm[0, c, r, :] = l
            ctx_acc[bh] = ctx_t
            lacc_ref[bh, :] = l
            if r == GRP - 1:
                start_hop(0, c)
                if c > 0:
                    wait_arrival(0, c - 1)
                    start_hop(1, c - 1)
                    combine(1, c - 1)

        wait_arrival(0, C - 1)
        start_hop(1, C - 1)
        combine(1, C - 1)
        for c in range(C):
            wait_arrival(1, c)
            start_hop(2, c)
            combine(2, c)
        for c in range(C):
            wait_arrival(2, c)
            combine(3, c)

        wo_bf = wo_ref[...].astype(jnp.bfloat16)
        for b in range(B):
            rows = pl.ds(b * HQ, HQ)
            l_b = lacc_ref[rows, :]
            ctxn = ctx_acc[rows] / l_b[:, None, :]
            cc = ctxn.astype(jnp.bfloat16).reshape(HQ * DH, SQ)
            out_ref[b] = lax.dot_general(
                cc, wo_bf, (((0,), (0,)), ((), ())),
                preferred_element_type=jnp.float32,
            )

        for rdma in started:
            rdma.wait_send()

    return pl.pallas_call(
        body,
        out_shape=jax.ShapeDtypeStruct((B, SQ, d_model), jnp.float32),
        in_specs=[pl.BlockSpec(memory_space=pltpu.VMEM)] * 5,
        out_specs=pl.BlockSpec(memory_space=pltpu.VMEM),
        scratch_shapes=[
            pltpu.VMEM((N_DEV, BH, DH, SQ), jnp.bfloat16),
            pltpu.VMEM((N_DEV, C, GRP, SQ), jnp.float32),
            pltpu.VMEM((BH, SQ), jnp.float32),
            pltpu.VMEM((BH, DH, SQ), jnp.float32),
            pltpu.SemaphoreType.DMA((N_DEV - 1, C)),
            pltpu.SemaphoreType.DMA((N_DEV - 1, C)),
            pltpu.SemaphoreType.DMA((N_DEV - 1, C)),
            pltpu.SemaphoreType.DMA((N_DEV - 1, C)),
        ],
        compiler_params=pltpu.CompilerParams(collective_id=0),
    )(x, Wq, K_ext, V_ext, Wo)


# device time: 51075 ns/iter; 1.2534x vs baseline; 1.0354x over previous
import jax
import jax.numpy as jnp
from jax import lax
from jax.experimental import pallas as pl
from jax.experimental.pallas import tpu as pltpu

N_DEV = 4
B = 2
SQ = 512
SKV_SHARD = 512
HQ = 8
DH = 64
BH = B * HQ
C = 8
GRP = BH // C


def kernel(x, Wq, K_ext, V_ext, Wo):
    d_model = x.shape[-1]
    K_ext = K_ext.reshape(B, SKV_SHARD, HQ * DH)
    V_ext = V_ext.reshape(B, SKV_SHARD, HQ * DH)

    def body(
        x_ref, wq_ref, k_ref, v_ref, wo_ref, out_ref,
        ctx_comm, stats_comm, lacc_ref, ctx_acc,
        ctx_send_sems, ctx_recv_sems, st_send_sems, st_recv_sems,
    ):
        my_pos = lax.axis_index("i")
        left = (my_pos - 1) % N_DEV
        right = (my_pos + 1) % N_DEV

        barrier_sem = pltpu.get_barrier_semaphore()
        for nbr in [left, right]:
            pl.semaphore_signal(
                barrier_sem, inc=1,
                device_id=(nbr,), device_id_type=pl.DeviceIdType.MESH,
            )
        pl.semaphore_wait(barrier_sem, 2)

        started = []

        def make_chunk_rdmas(h, c):
            rc = pltpu.make_async_remote_copy(
                src_ref=ctx_comm.at[h, pl.ds(GRP * c, GRP)],
                dst_ref=ctx_comm.at[h + 1, pl.ds(GRP * c, GRP)],
                send_sem=ctx_send_sems.at[h, c],
                recv_sem=ctx_recv_sems.at[h, c],
                device_id=(right,),
                device_id_type=pl.DeviceIdType.MESH,
            )
            rs = pltpu.make_async_remote_copy(
                src_ref=stats_comm.at[h, c],
                dst_ref=stats_comm.at[h + 1, c],
                send_sem=st_send_sems.at[h, c],
                recv_sem=st_recv_sems.at[h, c],
                device_id=(right,),
                device_id_type=pl.DeviceIdType.MESH,
            )
            return rc, rs

        def start_hop(h, c):
            rc, rs = make_chunk_rdmas(h, c)
            rc.start()
            rs.start()
            started.extend((rc, rs))

        def wait_arrival(h, c):
            rc, rs = make_chunk_rdmas(h, c)
            rc.wait_recv()
            rs.wait_recv()

        def combine(slot, c):
            rows = pl.ds(GRP * c, GRP)
            lacc_ref[rows, :] = lacc_ref[rows, :] + stats_comm[slot, c]
            ctx_acc[rows] = (
                ctx_acc[rows] + ctx_comm[slot, rows].astype(jnp.float32)
            )

        kv_off = my_pos * SKV_SHARD
        qi = lax.broadcasted_iota(jnp.int32, (SQ, SKV_SHARD), 0)
        kj = lax.broadcasted_iota(jnp.int32, (SQ, SKV_SHARD), 1) + kv_off
        mask = (jnp.abs(qi - kj) <= 128) | (kj < 32) | (qi < 32)
        bias = jnp.where(mask, 0.0, -1e9).astype(jnp.float32)

        for bh in range(BH):
            b, h = divmod(bh, HQ)
            hs = slice(h * DH, (h + 1) * DH)
            if h == 0:
                q_all = (jnp.dot(
                    x_ref[b].astype(jnp.bfloat16),
                    wq_ref[...].astype(jnp.bfloat16),
                    preferred_element_type=jnp.float32,
                ) * 0.125).astype(jnp.bfloat16)
                k_all = k_ref[b].astype(jnp.bfloat16)
                v_all = v_ref[b].astype(jnp.bfloat16)
            q = q_all[:, hs]
            k = k_all[:, hs]
            v = v_all[:, hs]
            s = lax.dot_general(
                q, k, (((1,), (1,)), ((), ())),
                preferred_element_type=jnp.float32,
            )
            p = jnp.exp(s + bias)
            l = jnp.sum(p, axis=1)
            ctx_t = lax.dot_general(
                v, p.astype(jnp.bfloat16), (((0,), (1,)), ((), ())),
                preferred_element_type=jnp.float32,
            )
            c, r = divmod(bh, GRP)
            ctx_comm[0, bh] = ctx_t.astype(jnp.bfloat16)
            stats_comm[0, c, r, :] = l
            ctx_acc[bh] = ctx_t
            lacc_ref[bh, :] = l
            if r == GRP - 1:
                start_hop(0, c)
                if c > 0:
                    wait_arrival(0, c - 1)
                    start_hop(1, c - 1)
                    combine(1, c - 1)

        wait_arrival(0, C - 1)
        start_hop(1, C - 1)
        combine(1, C - 1)
        for c in range(C):
            wait_arrival(1, c)
            start_hop(2, c)
            combine(2, c)
        for c in range(C):
            wait_arrival(2, c)
            combine(3, c)

        wo_bf = wo_ref[...].astype(jnp.bfloat16)
        for b in range(B):
            rows = pl.ds(b * HQ, HQ)
            l_b = lacc_ref[rows, :]
            ctxn = ctx_acc[rows] / l_b[:, None, :]
            cc = ctxn.astype(jnp.bfloat16).reshape(HQ * DH, SQ)
            out_ref[b] = lax.dot_general(
                cc, wo_bf, (((0,), (0,)), ((), ())),
                preferred_element_type=jnp.float32,
            )

        for rdma in started:
            rdma.wait_send()

    return pl.pallas_call(
        body,
        out_shape=jax.ShapeDtypeStruct((B, SQ, d_model), jnp.float32),
        in_specs=[pl.BlockSpec(memory_space=pltpu.VMEM)] * 5,
        out_specs=pl.BlockSpec(memory_space=pltpu.VMEM),
        scratch_shapes=[
            pltpu.VMEM((N_DEV, BH, DH, SQ), jnp.bfloat16),
            pltpu.VMEM((N_DEV, C, GRP, SQ), jnp.float32),
            pltpu.VMEM((BH, SQ), jnp.float32),
            pltpu.VMEM((BH, DH, SQ), jnp.float32),
            pltpu.SemaphoreType.DMA((N_DEV - 1, C)),
            pltpu.SemaphoreType.DMA((N_DEV - 1, C)),
            pltpu.SemaphoreType.DMA((N_DEV - 1, C)),
            pltpu.SemaphoreType.DMA((N_DEV - 1, C)),
        ],
        compiler_params=pltpu.CompilerParams(collective_id=0),
    )(x, Wq, K_ext, V_ext, Wo)
